# baseline (device time: 258657 ns/iter reference)
import jax
import jax.numpy as jnp
from jax import lax
from jax.experimental import pallas as pl
from jax.experimental.pallas import tpu as pltpu

B = 2
H = 256
W = 256
C = 128
GLOBAL_SPATIAL = 512.0 * 512.0
EPS = 1e-5
CHUNK = 32
NCH = H // CHUNK
MESH = pl.DeviceIdType.MESH


def kernel(x, k, Wp):
    def body(x_hbm, k_ref, w_ref, o_hbm,
             xb, padded, rowbuf, colbuf, colstage,
             stat_loc, stat_rx, stat_xsum, stat_ry,
             load_sem, store_sem,
             sx_send, sx_recv, sy_send, sy_recv,
             row_send, row_recv, col_send, col_recv):
        my_x = lax.axis_index("x")
        my_y = lax.axis_index("y")
        x_nbr = (1 - my_x, my_y)
        y_nbr = (my_x, 1 - my_y)

        bar = pltpu.get_barrier_semaphore()
        for nbr in (x_nbr, y_nbr):
            pl.semaphore_signal(bar, inc=1, device_id=nbr,
                                device_id_type=MESH)
        pl.semaphore_wait(bar, 2)

        wb = w_ref[...].astype(jnp.bfloat16)
        kv = k_ref[...]
        ksum = jnp.sum(kv, axis=(0, 1)).reshape(1, 1, C)

        for b in range(B):
            load = pltpu.make_async_copy(x_hbm.at[b], xb, load_sem)
            load.start()
            load.wait()

            send_row = 255 * (1 - my_x)
            row_rdma = pltpu.make_async_remote_copy(
                src_ref=xb.at[send_row],
                dst_ref=rowbuf.at[b],
                send_sem=row_send.at[b],
                recv_sem=row_recv.at[b],
                device_id=x_nbr,
                device_id_type=MESH,
            )
            row_rdma.start()

            def stats_step(i, carry):
                s1, s2 = carry
                v = xb[pl.ds(i * CHUNK, CHUNK), :, :]
                padded[pl.ds(1 + i * CHUNK, CHUNK), 1:W + 1, :] = (
                    v.astype(jnp.bfloat16))
                return s1 + jnp.sum(v, axis=(0, 1)), s2 + jnp.sum(v * v, axis=(0, 1))

            s1, s2 = lax.fori_loop(
                0, NCH, stats_step,
                (jnp.zeros((C,), jnp.float32), jnp.zeros((C,), jnp.float32)))
            stat_loc[0:1, :] = s1.reshape(1, C)
            stat_loc[1:2, :] = s2.reshape(1, C)

            sx = pltpu.make_async_remote_copy(
                src_ref=stat_loc, dst_ref=stat_rx.at[b],
                send_sem=sx_send.at[b], recv_sem=sx_recv.at[b],
                device_id=x_nbr, device_id_type=MESH)
            sx.start()

            row_rdma.wait()
            hrow = rowbuf[b].reshape(1, W, C).astype(jnp.bfloat16)

            @pl.when(my_x == 0)
            def _():
                padded[0:1, 1:W + 1, :] = padded[1:2, 1:W + 1, :]
                padded[H + 1:H + 2, 1:W + 1, :] = hrow

            @pl.when(my_x == 1)
            def _():
                padded[0:1, 1:W + 1, :] = hrow
                padded[H + 1:H + 2, 1:W + 1, :] = padded[H:H + 1, 1:W + 1, :]

            @pl.when(my_y == 0)
            def _():
                colstage[...] = padded[:, W:W + 1, :]
            @pl.when(my_y == 1)
            def _():
                colstage[...] = padded[:, 1:2, :]
            col_rdma = pltpu.make_async_remote_copy(
                src_ref=colstage,
                dst_ref=colbuf.at[b],
                send_sem=col_send.at[b], recv_sem=col_recv.at[b],
                device_id=y_nbr, device_id_type=MESH)
            col_rdma.start()

            sx.wait()
            stat_xsum[...] = stat_loc[...] + stat_rx[b]
            sy = pltpu.make_async_remote_copy(
                src_ref=stat_xsum, dst_ref=stat_ry.at[b],
                send_sem=sy_send.at[b], recv_sem=sy_recv.at[b],
                device_id=y_nbr, device_id_type=MESH)
            sy.start()

            col_rdma.wait()

            @pl.when(my_y == 0)
            def _():
                padded[:, 0:1, :] = padded[:, 1:2, :]
                padded[:, W + 1:W + 2, :] = colbuf[b]

            @pl.when(my_y == 1)
            def _():
                padded[:, 0:1, :] = colbuf[b]
                padded[:, W + 1:W + 2, :] = padded[:, W:W + 1, :]

            sy.wait()
            tot = stat_xsum[...] + stat_ry[b]
            mean = (tot[0:1, :] * (1.0 / GLOBAL_SPATIAL)).reshape(1, 1, C)
            ex2 = (tot[1:2, :] * (1.0 / GLOBAL_SPATIAL)).reshape(1, 1, C)
            rstd = lax.rsqrt(ex2 - mean * mean + EPS)
            k2 = (kv * rstd).astype(jnp.bfloat16)
            off2 = (mean * ksum * rstd).astype(jnp.bfloat16)

            def conv_step(i, _):
                r0 = i * CHUNK
                acc = None
                for di in range(3):
                    for dj in range(3):
                        tap = padded[pl.ds(r0 + di, CHUNK), dj:dj + W, :]
                        t = tap * k2[di, dj].reshape(1, 1, C)
                        acc = t if acc is None else acc + t
                acc = acc - off2
                a = acc * jax.nn.sigmoid(acc)
                proj = jnp.dot(
                    a.reshape(CHUNK * W, C), wb,
                    preferred_element_type=jnp.float32)
                xb[pl.ds(r0, CHUNK), :, :] = (
                    xb[pl.ds(r0, CHUNK), :, :] + proj.reshape(CHUNK, W, C))
                return 0

            lax.fori_loop(0, NCH, conv_step, 0)

            store = pltpu.make_async_copy(xb, o_hbm.at[b], store_sem)
            store.start()
            store.wait()

    return pl.pallas_call(
        body,
        out_shape=jax.ShapeDtypeStruct((B, H, W, C), jnp.float32),
        in_specs=[
            pl.BlockSpec(memory_space=pl.ANY),
            pl.BlockSpec(memory_space=pltpu.VMEM),
            pl.BlockSpec(memory_space=pltpu.VMEM),
        ],
        out_specs=pl.BlockSpec(memory_space=pl.ANY),
        scratch_shapes=[
            pltpu.VMEM((H, W, C), jnp.float32),
            pltpu.VMEM((H + 2, W + 2, C), jnp.bfloat16),
            pltpu.VMEM((B, W, C), jnp.float32),
            pltpu.VMEM((B, H + 2, 1, C), jnp.bfloat16),
            pltpu.VMEM((H + 2, 1, C), jnp.bfloat16),
            pltpu.VMEM((2, C), jnp.float32),
            pltpu.VMEM((B, 2, C), jnp.float32),
            pltpu.VMEM((2, C), jnp.float32),
            pltpu.VMEM((B, 2, C), jnp.float32),
            pltpu.SemaphoreType.DMA,
            pltpu.SemaphoreType.DMA,
            pltpu.SemaphoreType.DMA((B,)),
            pltpu.SemaphoreType.DMA((B,)),
            pltpu.SemaphoreType.DMA((B,)),
            pltpu.SemaphoreType.DMA((B,)),
            pltpu.SemaphoreType.DMA((B,)),
            pltpu.SemaphoreType.DMA((B,)),
            pltpu.SemaphoreType.DMA((B,)),
            pltpu.SemaphoreType.DMA((B,)),
        ],
        compiler_params=pltpu.CompilerParams(
            collective_id=0,
            vmem_limit_bytes=60 * 1024 * 1024,
        ),
    )(x, k, Wp)


# device time: 211818 ns/iter; 1.2211x vs baseline; 1.2211x over previous
import jax
import jax.numpy as jnp
from jax import lax
from jax.experimental import pallas as pl
from jax.experimental.pallas import tpu as pltpu

B = 2
H = 256
W = 256
C = 128
GLOBAL_SPATIAL = 512.0 * 512.0
EPS = 1e-5
CHUNK = 32
NCH = H // CHUNK
MESH = pl.DeviceIdType.MESH


def kernel(x, k, Wp):
    def body(x_hbm, k_ref, w_ref, o_hbm,
             xb, padded, rowbuf, colbuf, colstage,
             stat_loc, stat_rx, stat_xsum, stat_ry,
             load_sem, store_sem,
             sx_send, sx_recv, sy_send, sy_recv,
             row_send, row_recv, col_send, col_recv):
        my_x = lax.axis_index("x")
        my_y = lax.axis_index("y")
        x_nbr = (1 - my_x, my_y)
        y_nbr = (my_x, 1 - my_y)

        bar = pltpu.get_barrier_semaphore()
        for nbr in (x_nbr, y_nbr):
            pl.semaphore_signal(bar, inc=1, device_id=nbr,
                                device_id_type=MESH)
        pl.semaphore_wait(bar, 2)

        wb = w_ref[...].astype(jnp.bfloat16)
        kv = k_ref[...]
        ksum = jnp.sum(kv, axis=(0, 1)).reshape(1, 1, C)

        for b in range(B):
            load = pltpu.make_async_copy(x_hbm.at[b], xb, load_sem)
            load.start()
            load.wait()

            send_row = 255 * (1 - my_x)
            row_rdma = pltpu.make_async_remote_copy(
                src_ref=xb.at[send_row],
                dst_ref=rowbuf.at[b],
                send_sem=row_send.at[b],
                recv_sem=row_recv.at[b],
                device_id=x_nbr,
                device_id_type=MESH,
            )
            row_rdma.start()

            def stats_step(i, carry):
                s1, s2 = carry
                v = xb[pl.ds(i * CHUNK, CHUNK), :, :]
                padded[pl.ds(1 + i * CHUNK, CHUNK), 1:W + 1, :] = (
                    v.astype(jnp.bfloat16))
                return s1 + jnp.sum(v, axis=(0, 1)), s2 + jnp.sum(v * v, axis=(0, 1))

            s1, s2 = lax.fori_loop(
                0, NCH, stats_step,
                (jnp.zeros((C,), jnp.float32), jnp.zeros((C,), jnp.float32)))
            stat_loc[0:1, :] = s1.reshape(1, C)
            stat_loc[1:2, :] = s2.reshape(1, C)

            sx = pltpu.make_async_remote_copy(
                src_ref=stat_loc, dst_ref=stat_rx.at[b],
                send_sem=sx_send.at[b], recv_sem=sx_recv.at[b],
                device_id=x_nbr, device_id_type=MESH)
            sx.start()

            row_rdma.wait()
            hrow = rowbuf[b].reshape(1, W, C).astype(jnp.bfloat16)

            @pl.when(my_x == 0)
            def _():
                padded[0:1, 1:W + 1, :] = padded[1:2, 1:W + 1, :]
                padded[H + 1:H + 2, 1:W + 1, :] = hrow

            @pl.when(my_x == 1)
            def _():
                padded[0:1, 1:W + 1, :] = hrow
                padded[H + 1:H + 2, 1:W + 1, :] = padded[H:H + 1, 1:W + 1, :]

            @pl.when(my_y == 0)
            def _():
                colstage[...] = padded[:, W:W + 1, :]
            @pl.when(my_y == 1)
            def _():
                colstage[...] = padded[:, 1:2, :]
            col_rdma = pltpu.make_async_remote_copy(
                src_ref=colstage,
                dst_ref=colbuf.at[b],
                send_sem=col_send.at[b], recv_sem=col_recv.at[b],
                device_id=y_nbr, device_id_type=MESH)
            col_rdma.start()

            sx.wait()
            stat_xsum[...] = stat_loc[...] + stat_rx[b]
            sy = pltpu.make_async_remote_copy(
                src_ref=stat_xsum, dst_ref=stat_ry.at[b],
                send_sem=sy_send.at[b], recv_sem=sy_recv.at[b],
                device_id=y_nbr, device_id_type=MESH)
            sy.start()

            col_rdma.wait()

            @pl.when(my_y == 0)
            def _():
                padded[:, 0:1, :] = padded[:, 1:2, :]
                padded[:, W + 1:W + 2, :] = colbuf[b]

            @pl.when(my_y == 1)
            def _():
                padded[:, 0:1, :] = colbuf[b]
                padded[:, W + 1:W + 2, :] = padded[:, W:W + 1, :]

            sy.wait()
            tot = stat_xsum[...] + stat_ry[b]
            mean = (tot[0:1, :] * (1.0 / GLOBAL_SPATIAL)).reshape(1, 1, C)
            ex2 = (tot[1:2, :] * (1.0 / GLOBAL_SPATIAL)).reshape(1, 1, C)
            rstd = lax.rsqrt(ex2 - mean * mean + EPS)
            k2 = kv * rstd
            off2 = mean * ksum * rstd

            def conv_step(i, _):
                r0 = i * CHUNK
                acc = None
                for dj in range(3):
                    s = padded[pl.ds(r0, CHUNK + 2),
                               dj:dj + W, :].astype(jnp.float32)
                    for di in range(3):
                        t = s[di:di + CHUNK] * k2[di, dj].reshape(1, 1, C)
                        acc = t if acc is None else acc + t
                acc = acc - off2
                a = acc * jax.nn.sigmoid(acc)
                proj = jnp.dot(
                    a.reshape(CHUNK * W, C).astype(jnp.bfloat16), wb,
                    preferred_element_type=jnp.float32)
                xb[pl.ds(r0, CHUNK), :, :] = (
                    xb[pl.ds(r0, CHUNK), :, :] + proj.reshape(CHUNK, W, C))
                return 0

            lax.fori_loop(0, NCH, conv_step, 0)

            store = pltpu.make_async_copy(xb, o_hbm.at[b], store_sem)
            store.start()
            store.wait()

    return pl.pallas_call(
        body,
        out_shape=jax.ShapeDtypeStruct((B, H, W, C), jnp.float32),
        in_specs=[
            pl.BlockSpec(memory_space=pl.ANY),
            pl.BlockSpec(memory_space=pltpu.VMEM),
            pl.BlockSpec(memory_space=pltpu.VMEM),
        ],
        out_specs=pl.BlockSpec(memory_space=pl.ANY),
        scratch_shapes=[
            pltpu.VMEM((H, W, C), jnp.float32),
            pltpu.VMEM((H + 2, W + 2, C), jnp.bfloat16),
            pltpu.VMEM((B, W, C), jnp.float32),
            pltpu.VMEM((B, H + 2, 1, C), jnp.bfloat16),
            pltpu.VMEM((H + 2, 1, C), jnp.bfloat16),
            pltpu.VMEM((2, C), jnp.float32),
            pltpu.VMEM((B, 2, C), jnp.float32),
            pltpu.VMEM((2, C), jnp.float32),
            pltpu.VMEM((B, 2, C), jnp.float32),
            pltpu.SemaphoreType.DMA,
            pltpu.SemaphoreType.DMA,
            pltpu.SemaphoreType.DMA((B,)),
            pltpu.SemaphoreType.DMA((B,)),
            pltpu.SemaphoreType.DMA((B,)),
            pltpu.SemaphoreType.DMA((B,)),
            pltpu.SemaphoreType.DMA((B,)),
            pltpu.SemaphoreType.DMA((B,)),
            pltpu.SemaphoreType.DMA((B,)),
            pltpu.SemaphoreType.DMA((B,)),
        ],
        compiler_params=pltpu.CompilerParams(
            collective_id=0,
            vmem_limit_bytes=60 * 1024 * 1024,
        ),
    )(x, k, Wp)


# device time: 177901 ns/iter; 1.4539x vs baseline; 1.1907x over previous
import jax
import jax.numpy as jnp
from jax import lax
from jax.experimental import pallas as pl
from jax.experimental.pallas import tpu as pltpu

B = 2
H = 256
W = 256
C = 128
GLOBAL_SPATIAL = 512.0 * 512.0
EPS = 1e-5
CHUNK = 32
NCH = H // CHUNK
MESH = pl.DeviceIdType.MESH


def kernel(x, k, Wp):
    def body(x_hbm, k_ref, w_ref, o_hbm,
             xb, padded, rowbuf, colbuf, colstage, rowstage,
             stat_loc, stat_rx, stat_xsum, stat_ry,
             load_sems, store_sems, row_load_sem,
             sx_send, sx_recv, sy_send, sy_recv,
             row_send, row_recv, col_send, col_recv):
        my_x = lax.axis_index("x")
        my_y = lax.axis_index("y")
        x_nbr = (1 - my_x, my_y)
        y_nbr = (my_x, 1 - my_y)

        bar = pltpu.get_barrier_semaphore()
        for nbr in (x_nbr, y_nbr):
            pl.semaphore_signal(bar, inc=1, device_id=nbr,
                                device_id_type=MESH)
        pl.semaphore_wait(bar, 2)

        wb = w_ref[...].astype(jnp.bfloat16)
        kv = k_ref[...]
        ksum = jnp.sum(kv, axis=(0, 1)).reshape(1, 1, C)

        for b in range(B):
            send_row = 255 * (1 - my_x)
            rload = pltpu.make_async_copy(
                x_hbm.at[b, send_row], rowstage, row_load_sem)
            rload.start()

            def load_issue(i, _):
                pltpu.make_async_copy(
                    x_hbm.at[b, pl.ds(i * CHUNK, CHUNK)],
                    xb.at[pl.ds(i * CHUNK, CHUNK)],
                    load_sems.at[i]).start()
                return 0

            lax.fori_loop(0, NCH, load_issue, 0)

            rload.wait()
            row_rdma = pltpu.make_async_remote_copy(
                src_ref=rowstage,
                dst_ref=rowbuf.at[b],
                send_sem=row_send.at[b],
                recv_sem=row_recv.at[b],
                device_id=x_nbr,
                device_id_type=MESH,
            )
            row_rdma.start()

            def stats_step(i, carry):
                s1, s2 = carry
                pltpu.make_async_copy(
                    x_hbm.at[b, pl.ds(i * CHUNK, CHUNK)],
                    xb.at[pl.ds(i * CHUNK, CHUNK)],
                    load_sems.at[i]).wait()
                v = xb[pl.ds(i * CHUNK, CHUNK), :, :]
                padded[pl.ds(1 + i * CHUNK, CHUNK), 1:W + 1, :] = (
                    v.astype(jnp.bfloat16))
                return s1 + jnp.sum(v, axis=(0, 1)), s2 + jnp.sum(v * v, axis=(0, 1))

            s1, s2 = lax.fori_loop(
                0, NCH, stats_step,
                (jnp.zeros((C,), jnp.float32), jnp.zeros((C,), jnp.float32)))
            stat_loc[0:1, :] = s1.reshape(1, C)
            stat_loc[1:2, :] = s2.reshape(1, C)

            sx = pltpu.make_async_remote_copy(
                src_ref=stat_loc, dst_ref=stat_rx.at[b],
                send_sem=sx_send.at[b], recv_sem=sx_recv.at[b],
                device_id=x_nbr, device_id_type=MESH)
            sx.start()

            row_rdma.wait()
            hrow = rowbuf[b].reshape(1, W, C).astype(jnp.bfloat16)

            @pl.when(my_x == 0)
            def _():
                padded[0:1, 1:W + 1, :] = padded[1:2, 1:W + 1, :]
                padded[H + 1:H + 2, 1:W + 1, :] = hrow

            @pl.when(my_x == 1)
            def _():
                padded[0:1, 1:W + 1, :] = hrow
                padded[H + 1:H + 2, 1:W + 1, :] = padded[H:H + 1, 1:W + 1, :]

            @pl.when(my_y == 0)
            def _():
                colstage[...] = padded[:, W:W + 1, :]
            @pl.when(my_y == 1)
            def _():
                colstage[...] = padded[:, 1:2, :]
            col_rdma = pltpu.make_async_remote_copy(
                src_ref=colstage,
                dst_ref=colbuf.at[b],
                send_sem=col_send.at[b], recv_sem=col_recv.at[b],
                device_id=y_nbr, device_id_type=MESH)
            col_rdma.start()

            sx.wait()
            stat_xsum[...] = stat_loc[...] + stat_rx[b]
            sy = pltpu.make_async_remote_copy(
                src_ref=stat_xsum, dst_ref=stat_ry.at[b],
                send_sem=sy_send.at[b], recv_sem=sy_recv.at[b],
                device_id=y_nbr, device_id_type=MESH)
            sy.start()

            col_rdma.wait()

            @pl.when(my_y == 0)
            def _():
                padded[:, 0:1, :] = padded[:, 1:2, :]
                padded[:, W + 1:W + 2, :] = colbuf[b]

            @pl.when(my_y == 1)
            def _():
                padded[:, 0:1, :] = colbuf[b]
                padded[:, W + 1:W + 2, :] = padded[:, W:W + 1, :]

            sy.wait()
            tot = stat_xsum[...] + stat_ry[b]
            mean = (tot[0:1, :] * (1.0 / GLOBAL_SPATIAL)).reshape(1, 1, C)
            ex2 = (tot[1:2, :] * (1.0 / GLOBAL_SPATIAL)).reshape(1, 1, C)
            rstd = lax.rsqrt(ex2 - mean * mean + EPS)
            k2 = kv * rstd
            off2 = mean * ksum * rstd

            def conv_step(i, _):
                r0 = i * CHUNK
                acc = None
                for dj in range(3):
                    s = padded[pl.ds(r0, CHUNK + 2),
                               dj:dj + W, :].astype(jnp.float32)
                    for di in range(3):
                        t = s[di:di + CHUNK] * k2[di, dj].reshape(1, 1, C)
                        acc = t if acc is None else acc + t
                acc = acc - off2
                a = acc * jax.nn.sigmoid(acc)
                proj = jnp.dot(
                    a.reshape(CHUNK * W, C).astype(jnp.bfloat16), wb,
                    preferred_element_type=jnp.float32)
                xb[pl.ds(r0, CHUNK), :, :] = (
                    xb[pl.ds(r0, CHUNK), :, :] + proj.reshape(CHUNK, W, C))
                pltpu.make_async_copy(
                    xb.at[pl.ds(r0, CHUNK)],
                    o_hbm.at[b, pl.ds(r0, CHUNK)],
                    store_sems.at[i]).start()
                return 0

            lax.fori_loop(0, NCH, conv_step, 0)

            def store_wait(i, _):
                pltpu.make_async_copy(
                    xb.at[pl.ds(i * CHUNK, CHUNK)],
                    o_hbm.at[b, pl.ds(i * CHUNK, CHUNK)],
                    store_sems.at[i]).wait()
                return 0

            lax.fori_loop(0, NCH, store_wait, 0)

    return pl.pallas_call(
        body,
        out_shape=jax.ShapeDtypeStruct((B, H, W, C), jnp.float32),
        in_specs=[
            pl.BlockSpec(memory_space=pl.ANY),
            pl.BlockSpec(memory_space=pltpu.VMEM),
            pl.BlockSpec(memory_space=pltpu.VMEM),
        ],
        out_specs=pl.BlockSpec(memory_space=pl.ANY),
        scratch_shapes=[
            pltpu.VMEM((H, W, C), jnp.float32),
            pltpu.VMEM((H + 2, W + 2, C), jnp.bfloat16),
            pltpu.VMEM((B, W, C), jnp.float32),
            pltpu.VMEM((B, H + 2, 1, C), jnp.bfloat16),
            pltpu.VMEM((H + 2, 1, C), jnp.bfloat16),
            pltpu.VMEM((W, C), jnp.float32),
            pltpu.VMEM((2, C), jnp.float32),
            pltpu.VMEM((B, 2, C), jnp.float32),
            pltpu.VMEM((2, C), jnp.float32),
            pltpu.VMEM((B, 2, C), jnp.float32),
            pltpu.SemaphoreType.DMA((NCH,)),
            pltpu.SemaphoreType.DMA((NCH,)),
            pltpu.SemaphoreType.DMA,
            pltpu.SemaphoreType.DMA((B,)),
            pltpu.SemaphoreType.DMA((B,)),
            pltpu.SemaphoreType.DMA((B,)),
            pltpu.SemaphoreType.DMA((B,)),
            pltpu.SemaphoreType.DMA((B,)),
            pltpu.SemaphoreType.DMA((B,)),
            pltpu.SemaphoreType.DMA((B,)),
            pltpu.SemaphoreType.DMA((B,)),
        ],
        compiler_params=pltpu.CompilerParams(
            collective_id=0,
            vmem_limit_bytes=60 * 1024 * 1024,
        ),
    )(x, k, Wp)
